# Initial kernel scaffold; baseline (speedup 1.0000x reference)
import jax
import jax.numpy as jnp
from jax.experimental import pallas as pl


def kernel(x, edge_index, W, b, loop_weight):
    raise NotImplementedError("write your pallas kernel here")


# speedup vs baseline: 3.1291x; 3.1291x over previous
import functools

import jax
import jax.numpy as jnp
from jax import lax
from jax.experimental import pallas as pl
from jax.experimental.pallas import tpu as pltpu
from jax.experimental.pallas import tpu_sc as plsc

N_NODES = 10000
N_EDGES = 160000
FEAT = 256
HALF = 128
FPAD = 144
NSC = 2
NSUB = 16
EPT = 10240
EPAD = NSUB * EPT
CHUNK = 128
NCHUNK = EPT // CHUNK
ZROWS = 632
AGG_ROWS = NSUB * ZROWS


def _sc_aggregate(xcat, src_p, dst_p, zrows):
  mesh = plsc.VectorSubcoreMesh(
      core_axis_name="c", subcore_axis_name="s",
      num_cores=NSC, num_subcores=NSUB)

  @functools.partial(
      pl.kernel,
      out_type=jax.ShapeDtypeStruct((NSC, AGG_ROWS, FPAD), jnp.float32),
      mesh=mesh,
      compiler_params=pltpu.CompilerParams(use_tc_tiling_on_sc=False),
      scratch_types=[
          pltpu.VMEM_SHARED((AGG_ROWS, FPAD), jnp.float32),
          pltpu.VMEM((NCHUNK, CHUNK), jnp.int32),
          pltpu.VMEM((NCHUNK, CHUNK), jnp.int32),
          pltpu.VMEM((CHUNK, FPAD), jnp.float32),
          pltpu.SemaphoreType.DMA,
      ],
  )
  def agg_kernel(xcat_hbm, src_hbm, dst_hbm, z_hbm, out_hbm,
                 acc, src_v, dst_v, rows0, sem0):
    c = lax.axis_index("c")
    s = lax.axis_index("s")

    pltpu.sync_copy(z_hbm, acc.at[pl.ds(s * ZROWS, ZROWS)])

    pltpu.sync_copy(src_hbm.at[c, s], src_v)
    pltpu.sync_copy(dst_hbm.at[s], dst_v)

    plsc.subcore_barrier()

    @pl.loop(0, NCHUNK)
    def _(j):
      pltpu.async_copy(xcat_hbm.at[src_v.at[j]], rows0, sem0).wait()
      pltpu.sync_copy(rows0, acc.at[dst_v.at[j]], add=True)

    plsc.subcore_barrier()
    pltpu.sync_copy(acc.at[pl.ds(s * ZROWS, ZROWS)],
                    out_hbm.at[c, pl.ds(s * ZROWS, ZROWS)])

  return agg_kernel(xcat, src_p, dst_p, zrows)


def _tc_body(x_ref, agg_ref, w_ref, lw_ref, b_ref, out_ref):
  a0 = agg_ref[0]
  a1 = agg_ref[1]
  deg = jnp.maximum(a0[:, HALF:HALF + 1], 1.0)
  inv = 1.0 / deg
  acc = jnp.dot(a0 * inv, w_ref[0], preferred_element_type=jnp.float32)
  acc = acc + jnp.dot(a1 * inv, w_ref[1], preferred_element_type=jnp.float32)
  acc = acc + jnp.dot(x_ref[...], lw_ref[...], preferred_element_type=jnp.float32)
  out_ref[...] = acc + b_ref[...]


def _tc_combine(x, aggdeg, w2p, lw, b2):
  nblk = 10
  blk = N_NODES // nblk
  return pl.pallas_call(
      _tc_body,
      grid=(nblk,),
      in_specs=[
          pl.BlockSpec((blk, FEAT), lambda i: (i, 0)),
          pl.BlockSpec((NSC, blk, FPAD), lambda i: (0, i, 0)),
          pl.BlockSpec((NSC, FPAD, FEAT), lambda i: (0, 0, 0)),
          pl.BlockSpec((FEAT, FEAT), lambda i: (0, 0)),
          pl.BlockSpec((1, FEAT), lambda i: (0, 0)),
      ],
      out_specs=pl.BlockSpec((blk, FEAT), lambda i: (i, 0)),
      out_shape=jax.ShapeDtypeStruct((N_NODES, FEAT), jnp.float32),
  )(x, aggdeg, w2p, lw, b2)


def kernel(x, edge_index, W, b, loop_weight):
  ei = edge_index.astype(jnp.int32)
  src = ei[0]
  dst = ei[1]
  pad = EPAD - N_EDGES
  src_p = jnp.concatenate([src, jnp.full((pad,), 2 * N_NODES, jnp.int32)])
  src_p = src_p.reshape(1, NSUB, NCHUNK, CHUNK)
  src_p = jnp.concatenate([src_p, src_p + N_NODES], axis=0)
  dst_p = jnp.concatenate([dst, jnp.full((pad,), N_NODES, jnp.int32)])
  dst_p = dst_p.reshape(NSUB, NCHUNK, CHUNK)

  ones = jnp.ones((N_NODES, 1), jnp.float32)
  zcols = jnp.zeros((N_NODES, FPAD - HALF - 1), jnp.float32)
  xe0 = jnp.concatenate([x[:, :HALF], ones, zcols], axis=1)
  xe1 = jnp.concatenate([x[:, HALF:], ones, zcols], axis=1)
  xcat = jnp.concatenate([xe0, xe1, jnp.zeros((8, FPAD), jnp.float32)], axis=0)
  zrows = jnp.zeros((ZROWS, FPAD), jnp.float32)

  aggdeg = _sc_aggregate(xcat, src_p, dst_p, zrows)

  w2p = jnp.zeros((NSC, FPAD, FEAT), jnp.float32)
  w2p = w2p.at[:, :HALF, :].set(W.reshape(NSC, HALF, FEAT))
  return _tc_combine(x, aggdeg, w2p, loop_weight, b.reshape(1, FEAT))


# speedup vs baseline: 3.2099x; 1.0258x over previous
import functools

import jax
import jax.numpy as jnp
from jax import lax
from jax.experimental import pallas as pl
from jax.experimental.pallas import tpu as pltpu
from jax.experimental.pallas import tpu_sc as plsc

N_NODES = 10000
N_EDGES = 160000
FEAT = 256
HALF = 128
FPAD = 144
NSC = 2
NSUB = 16
EPT = 10240
EPAD = NSUB * EPT
CHUNK = 64
NCHUNK = EPT // CHUNK
ZROWS = 632
AGG_ROWS = NSUB * ZROWS


def _sc_aggregate(xcat, src_p, dst_p, zrows):
  mesh = plsc.VectorSubcoreMesh(
      core_axis_name="c", subcore_axis_name="s",
      num_cores=NSC, num_subcores=NSUB)

  @functools.partial(
      pl.kernel,
      out_type=jax.ShapeDtypeStruct((NSC, AGG_ROWS, FPAD), jnp.float32),
      mesh=mesh,
      compiler_params=pltpu.CompilerParams(use_tc_tiling_on_sc=False),
      scratch_types=[
          pltpu.VMEM_SHARED((AGG_ROWS, FPAD), jnp.float32),
          pltpu.VMEM((NCHUNK, CHUNK), jnp.int32),
          pltpu.VMEM((NCHUNK, CHUNK), jnp.int32),
          pltpu.VMEM((CHUNK, FPAD), jnp.float32),
          pltpu.VMEM((CHUNK, FPAD), jnp.float32),
          pltpu.SemaphoreType.DMA,
          pltpu.SemaphoreType.DMA,
      ],
  )
  def agg_kernel(xcat_hbm, src_hbm, dst_hbm, z_hbm, out_hbm,
                 acc, src_v, dst_v, rows0, rows1, sem0, sem1):
    c = lax.axis_index("c")
    s = lax.axis_index("s")

    pltpu.sync_copy(z_hbm, acc.at[pl.ds(s * ZROWS, ZROWS)])

    pltpu.sync_copy(src_hbm.at[c, s], src_v)
    pltpu.sync_copy(dst_hbm.at[s], dst_v)

    plsc.subcore_barrier()

    pltpu.async_copy(xcat_hbm.at[src_v.at[0]], rows0, sem0)

    @pl.loop(0, NCHUNK, step=2)
    def _(j):
      pltpu.make_async_copy(xcat_hbm.at[src_v.at[j]], rows0, sem0).wait()
      pltpu.async_copy(xcat_hbm.at[src_v.at[j + 1]], rows1, sem1)
      pltpu.sync_copy(rows0, acc.at[dst_v.at[j]], add=True)
      pltpu.make_async_copy(xcat_hbm.at[src_v.at[j + 1]], rows1, sem1).wait()

      @pl.when(j + 2 < NCHUNK)
      def _():
        pltpu.async_copy(xcat_hbm.at[src_v.at[j + 2]], rows0, sem0)

      pltpu.sync_copy(rows1, acc.at[dst_v.at[j + 1]], add=True)

    plsc.subcore_barrier()
    pltpu.sync_copy(acc.at[pl.ds(s * ZROWS, ZROWS)],
                    out_hbm.at[c, pl.ds(s * ZROWS, ZROWS)])

  return agg_kernel(xcat, src_p, dst_p, zrows)


def _tc_body(x_ref, agg_ref, w_ref, lw_ref, b_ref, out_ref):
  a0 = agg_ref[0]
  a1 = agg_ref[1]
  deg = jnp.maximum(a0[:, HALF:HALF + 1], 1.0)
  inv = 1.0 / deg
  acc = jnp.dot(a0 * inv, w_ref[0], preferred_element_type=jnp.float32)
  acc = acc + jnp.dot(a1 * inv, w_ref[1], preferred_element_type=jnp.float32)
  acc = acc + jnp.dot(x_ref[...], lw_ref[...], preferred_element_type=jnp.float32)
  out_ref[...] = acc + b_ref[...]


def _tc_combine(x, aggdeg, w2p, lw, b2):
  nblk = 10
  blk = N_NODES // nblk
  return pl.pallas_call(
      _tc_body,
      grid=(nblk,),
      in_specs=[
          pl.BlockSpec((blk, FEAT), lambda i: (i, 0)),
          pl.BlockSpec((NSC, blk, FPAD), lambda i: (0, i, 0)),
          pl.BlockSpec((NSC, FPAD, FEAT), lambda i: (0, 0, 0)),
          pl.BlockSpec((FEAT, FEAT), lambda i: (0, 0)),
          pl.BlockSpec((1, FEAT), lambda i: (0, 0)),
      ],
      out_specs=pl.BlockSpec((blk, FEAT), lambda i: (i, 0)),
      out_shape=jax.ShapeDtypeStruct((N_NODES, FEAT), jnp.float32),
  )(x, aggdeg, w2p, lw, b2)


def kernel(x, edge_index, W, b, loop_weight):
  ei = edge_index.astype(jnp.int32)
  src = ei[0]
  dst = ei[1]
  pad = EPAD - N_EDGES
  src_p = jnp.concatenate([src, jnp.full((pad,), 2 * N_NODES, jnp.int32)])
  src_p = src_p.reshape(1, NSUB, NCHUNK, CHUNK)
  src_p = jnp.concatenate([src_p, src_p + N_NODES], axis=0)
  dst_p = jnp.concatenate([dst, jnp.full((pad,), N_NODES, jnp.int32)])
  dst_p = dst_p.reshape(NSUB, NCHUNK, CHUNK)

  ones = jnp.ones((N_NODES, 1), jnp.float32)
  zcols = jnp.zeros((N_NODES, FPAD - HALF - 1), jnp.float32)
  xe0 = jnp.concatenate([x[:, :HALF], ones, zcols], axis=1)
  xe1 = jnp.concatenate([x[:, HALF:], ones, zcols], axis=1)
  xcat = jnp.concatenate([xe0, xe1, jnp.zeros((8, FPAD), jnp.float32)], axis=0)
  zrows = jnp.zeros((ZROWS, FPAD), jnp.float32)

  aggdeg = _sc_aggregate(xcat, src_p, dst_p, zrows)

  w2p = jnp.zeros((NSC, FPAD, FEAT), jnp.float32)
  w2p = w2p.at[:, :HALF, :].set(W.reshape(NSC, HALF, FEAT))
  return _tc_combine(x, aggdeg, w2p, loop_weight, b.reshape(1, FEAT))


# speedup vs baseline: 4.9645x; 1.5466x over previous
import functools

import jax
import jax.numpy as jnp
from jax import lax
from jax.experimental import pallas as pl
from jax.experimental.pallas import tpu as pltpu
from jax.experimental.pallas import tpu_sc as plsc

N_NODES = 10000
N_EDGES = 160000
FEAT = 256
FPAD = 272
NSC = 2
NSUB = 16
NHALF = N_NODES // NSC
EPT = 10240
EPAD = NSUB * EPT
CHUNK = 32
IDXB = EPT + 2 * CHUNK + 16
ZROWS = 316
AGG_ROWS = NSUB * ZROWS
DUMMY_SRC = N_NODES
DUMMY_DST = NHALF


def _sc_aggregate(xfull, src_p, dst_p, zrows):
  mesh = plsc.VectorSubcoreMesh(
      core_axis_name="c", subcore_axis_name="s",
      num_cores=NSC, num_subcores=NSUB)

  @functools.partial(
      pl.kernel,
      out_type=jax.ShapeDtypeStruct((NSC, AGG_ROWS, FPAD), jnp.float32),
      mesh=mesh,
      compiler_params=pltpu.CompilerParams(
          use_tc_tiling_on_sc=False, needs_layout_passes=False),
      scratch_types=[
          pltpu.VMEM_SHARED((AGG_ROWS, FPAD), jnp.float32),
          pltpu.VMEM((IDXB,), jnp.int32),
          pltpu.VMEM((IDXB,), jnp.int32),
          pltpu.VMEM((CHUNK, FPAD), jnp.float32),
          pltpu.VMEM((CHUNK, FPAD), jnp.float32),
          pltpu.SemaphoreType.DMA,
          pltpu.SemaphoreType.DMA,
      ],
  )
  def agg_kernel(xfull_hbm, src_hbm, dst_hbm, z_hbm, out_hbm,
                 acc, srcb, dstb, rows0, rows1, sem0, sem1):
    c = lax.axis_index("c")
    s = lax.axis_index("s")
    lo = c * NHALF

    pltpu.sync_copy(z_hbm, acc.at[pl.ds(s * ZROWS, ZROWS)])

    pltpu.sync_copy(src_hbm.at[s], srcb.at[pl.ds(0, EPT)])
    pltpu.sync_copy(dst_hbm.at[s], dstb.at[pl.ds(0, EPT)])

    @pl.loop(0, EPT // 16, init_carry=jnp.int32(0))
    def _filter(i, off):
      sl = pl.ds(i * 16, 16)
      d = dstb[sl]
      sv = srcb[sl]
      msk = (d >= lo) & (d < lo + NHALF)
      plsc.store_compressed(dstb.at[pl.ds(off, 16)], d - lo, mask=msk)
      plsc.store_compressed(srcb.at[pl.ds(off, 16)], sv, mask=msk)
      cnt = plsc.all_reduce_population_count(msk)
      return off + cnt[0]

    m = _filter
    base = pl.multiple_of((m // 16) * 16, 16)
    lanes = lax.iota(jnp.int32, 16)
    keep = lanes < (m - base)
    bsl = pl.ds(base, 16)
    srcb[bsl] = jnp.where(keep, srcb[bsl], DUMMY_SRC)
    dstb[bsl] = jnp.where(keep, dstb[bsl], DUMMY_DST)
    for k in range(1, 2 * CHUNK // 16 + 1):
      sl = pl.ds(base + k * 16, 16)
      srcb[sl] = jnp.full((16,), DUMMY_SRC, jnp.int32)
      dstb[sl] = jnp.full((16,), DUMMY_DST, jnp.int32)
    nch = jnp.maximum(2 * ((m + 2 * CHUNK - 1) // (2 * CHUNK)), 2)

    plsc.subcore_barrier()

    pltpu.async_copy(xfull_hbm.at[srcb.at[pl.ds(0, CHUNK)]], rows0, sem0)

    def _scatter(j, rows):
      for k in range(CHUNK // 16):
        dv = dstb[pl.ds(j * CHUNK + k * 16, 16)]
        pltpu.sync_copy(rows.at[pl.ds(k * 16, 16)], acc.at[dv], add=True)

    @pl.loop(0, nch, step=2)
    def _(j):
      s0 = srcb.at[pl.ds(j * CHUNK, CHUNK)]
      s1 = srcb.at[pl.ds((j + 1) * CHUNK, CHUNK)]
      pltpu.make_async_copy(xfull_hbm.at[s0], rows0, sem0).wait()
      pltpu.async_copy(xfull_hbm.at[s1], rows1, sem1)
      _scatter(j, rows0)
      pltpu.make_async_copy(xfull_hbm.at[s1], rows1, sem1).wait()

      @pl.when(j + 2 < nch)
      def _():
        s2 = srcb.at[pl.ds((j + 2) * CHUNK, CHUNK)]
        pltpu.async_copy(xfull_hbm.at[s2], rows0, sem0)

      _scatter(j + 1, rows1)

    plsc.subcore_barrier()
    pltpu.sync_copy(acc.at[pl.ds(s * ZROWS, ZROWS)],
                    out_hbm.at[c, pl.ds(s * ZROWS, ZROWS)])

  return agg_kernel(xfull, src_p, dst_p, zrows)


def _tc_body(x_ref, agg_ref, w_ref, lw_ref, b_ref, out_ref):
  a = agg_ref[0]
  deg = jnp.maximum(a[:, FEAT:FEAT + 1], 1.0)
  inv = 1.0 / deg
  acc = jnp.dot(a * inv, w_ref[...], preferred_element_type=jnp.float32)
  acc = acc + jnp.dot(x_ref[...], lw_ref[...], preferred_element_type=jnp.float32)
  out_ref[...] = acc + b_ref[...]


def _tc_combine(x, agg, w2p, lw, b2):
  nblk = 10
  blk = N_NODES // nblk
  bph = NHALF // blk
  return pl.pallas_call(
      _tc_body,
      grid=(nblk,),
      in_specs=[
          pl.BlockSpec((blk, FEAT), lambda i: (i, 0)),
          pl.BlockSpec((1, blk, FPAD), lambda i: (i // bph, i % bph, 0)),
          pl.BlockSpec((FPAD, FEAT), lambda i: (0, 0)),
          pl.BlockSpec((FEAT, FEAT), lambda i: (0, 0)),
          pl.BlockSpec((1, FEAT), lambda i: (0, 0)),
      ],
      out_specs=pl.BlockSpec((blk, FEAT), lambda i: (i, 0)),
      out_shape=jax.ShapeDtypeStruct((N_NODES, FEAT), jnp.float32),
  )(x, agg, w2p, lw, b2)


def kernel(x, edge_index, W, b, loop_weight):
  ei = edge_index.astype(jnp.int32)
  src = ei[0]
  dst = ei[1]
  pad = EPAD - N_EDGES
  src_p = jnp.concatenate([src, jnp.full((pad,), DUMMY_SRC, jnp.int32)])
  src_p = src_p.reshape(NSUB, EPT)
  dst_p = jnp.concatenate([dst, jnp.full((pad,), N_NODES, jnp.int32)])
  dst_p = dst_p.reshape(NSUB, EPT)

  ones = jnp.ones((N_NODES, 1), jnp.float32)
  zcols = jnp.zeros((N_NODES, FPAD - FEAT - 1), jnp.float32)
  xfull = jnp.concatenate([x, ones, zcols], axis=1)
  xfull = jnp.concatenate([xfull, jnp.zeros((8, FPAD), jnp.float32)], axis=0)
  zrows = jnp.zeros((ZROWS, FPAD), jnp.float32)

  agg = _sc_aggregate(xfull, src_p, dst_p, zrows)

  w2p = jnp.zeros((FPAD, FEAT), jnp.float32).at[:FEAT, :].set(W)
  return _tc_combine(x, agg, w2p, loop_weight, b.reshape(1, FEAT))


# speedup vs baseline: 6.1934x; 1.2475x over previous
import functools

import jax
import jax.numpy as jnp
from jax import lax
from jax.experimental import pallas as pl
from jax.experimental.pallas import tpu as pltpu
from jax.experimental.pallas import tpu_sc as plsc

N_NODES = 10000
N_EDGES = 160000
FEAT = 256
FPAD = 272
NSC = 2
NSUB = 16
NHALF = N_NODES // NSC
EPT = 10240
EPAD = NSUB * EPT
CHUNK = 16
NBUF = 4
IDXB = EPT + NBUF * CHUNK + 16
ZROWS = 316
AGG_ROWS = NSUB * ZROWS
DUMMY_SRC = N_NODES
DUMMY_DST = NHALF


def _sc_aggregate(xfull, src_p, dst_p, zrows):
  mesh = plsc.VectorSubcoreMesh(
      core_axis_name="c", subcore_axis_name="s",
      num_cores=NSC, num_subcores=NSUB)

  @functools.partial(
      pl.kernel,
      out_type=jax.ShapeDtypeStruct((NSC, AGG_ROWS, FPAD), jnp.float32),
      mesh=mesh,
      compiler_params=pltpu.CompilerParams(
          use_tc_tiling_on_sc=False, needs_layout_passes=False),
      scratch_types=[
          pltpu.VMEM_SHARED((AGG_ROWS, FPAD), jnp.float32),
          pltpu.VMEM((IDXB,), jnp.int32),
          pltpu.VMEM((IDXB,), jnp.int32),
          [pltpu.VMEM((CHUNK, FPAD), jnp.float32)] * NBUF,
          [pltpu.SemaphoreType.DMA] * NBUF,
      ],
  )
  def agg_kernel(xfull_hbm, src_hbm, dst_hbm, z_hbm, out_hbm,
                 acc, srcb, dstb, rows, sems):
    c = lax.axis_index("c")
    s = lax.axis_index("s")
    lo = c * NHALF

    pltpu.sync_copy(z_hbm, acc.at[pl.ds(s * ZROWS, ZROWS)])

    pltpu.sync_copy(src_hbm.at[s], srcb.at[pl.ds(0, EPT)])
    pltpu.sync_copy(dst_hbm.at[s], dstb.at[pl.ds(0, EPT)])

    @pl.loop(0, EPT // 16, init_carry=jnp.int32(0))
    def _filter(i, off):
      sl = pl.ds(i * 16, 16)
      d = dstb[sl]
      sv = srcb[sl]
      msk = (d >= lo) & (d < lo + NHALF)
      plsc.store_compressed(dstb.at[pl.ds(off, 16)], d - lo, mask=msk)
      plsc.store_compressed(srcb.at[pl.ds(off, 16)], sv, mask=msk)
      cnt = plsc.all_reduce_population_count(msk)
      return off + cnt[0]

    m = _filter
    base = pl.multiple_of((m // 16) * 16, 16)
    lanes = lax.iota(jnp.int32, 16)
    keep = lanes < (m - base)
    bsl = pl.ds(base, 16)
    srcb[bsl] = jnp.where(keep, srcb[bsl], DUMMY_SRC)
    dstb[bsl] = jnp.where(keep, dstb[bsl], DUMMY_DST)
    for k in range(1, NBUF * CHUNK // 16 + 1):
      sl = pl.ds(base + k * 16, 16)
      srcb[sl] = jnp.full((16,), DUMMY_SRC, jnp.int32)
      dstb[sl] = jnp.full((16,), DUMMY_DST, jnp.int32)
    grp = NBUF * CHUNK
    nch = jnp.maximum(NBUF * ((m + grp - 1) // grp), NBUF)

    plsc.subcore_barrier()

    def _gather(t, k):
      sl = srcb.at[pl.ds(t * CHUNK, CHUNK)]
      pltpu.async_copy(xfull_hbm.at[sl], rows[k], sems[k])

    for k in range(NBUF - 1):
      _gather(k, k)

    @pl.loop(0, nch, step=NBUF)
    def _(j):
      for k in range(NBUF):
        t = j + k
        sl = srcb.at[pl.ds(t * CHUNK, CHUNK)]
        pltpu.make_async_copy(xfull_hbm.at[sl], rows[k], sems[k]).wait()

        @pl.when(t + NBUF - 1 < nch)
        def _():
          _gather(t + NBUF - 1, (k + NBUF - 1) % NBUF)

        dv = dstb[pl.ds(t * CHUNK, 16)]
        pltpu.sync_copy(rows[k], acc.at[dv], add=True)

    plsc.subcore_barrier()
    pltpu.sync_copy(acc.at[pl.ds(s * ZROWS, ZROWS)],
                    out_hbm.at[c, pl.ds(s * ZROWS, ZROWS)])

  return agg_kernel(xfull, src_p, dst_p, zrows)


def _tc_body(x_ref, agg_ref, w_ref, lw_ref, b_ref, out_ref):
  a = agg_ref[0]
  deg = jnp.maximum(a[:, FEAT:FEAT + 1], 1.0)
  inv = 1.0 / deg
  acc = jnp.dot(a * inv, w_ref[...], preferred_element_type=jnp.float32)
  acc = acc + jnp.dot(x_ref[...], lw_ref[...], preferred_element_type=jnp.float32)
  out_ref[...] = acc + b_ref[...]


def _tc_combine(x, agg, w2p, lw, b2):
  nblk = 10
  blk = N_NODES // nblk
  bph = NHALF // blk
  return pl.pallas_call(
      _tc_body,
      grid=(nblk,),
      in_specs=[
          pl.BlockSpec((blk, FEAT), lambda i: (i, 0)),
          pl.BlockSpec((1, blk, FPAD), lambda i: (i // bph, i % bph, 0)),
          pl.BlockSpec((FPAD, FEAT), lambda i: (0, 0)),
          pl.BlockSpec((FEAT, FEAT), lambda i: (0, 0)),
          pl.BlockSpec((1, FEAT), lambda i: (0, 0)),
      ],
      out_specs=pl.BlockSpec((blk, FEAT), lambda i: (i, 0)),
      out_shape=jax.ShapeDtypeStruct((N_NODES, FEAT), jnp.float32),
  )(x, agg, w2p, lw, b2)


def kernel(x, edge_index, W, b, loop_weight):
  ei = edge_index.astype(jnp.int32)
  src = ei[0]
  dst = ei[1]
  pad = EPAD - N_EDGES
  src_p = jnp.concatenate([src, jnp.full((pad,), DUMMY_SRC, jnp.int32)])
  src_p = src_p.reshape(NSUB, EPT)
  dst_p = jnp.concatenate([dst, jnp.full((pad,), N_NODES, jnp.int32)])
  dst_p = dst_p.reshape(NSUB, EPT)

  ones = jnp.ones((N_NODES, 1), jnp.float32)
  zcols = jnp.zeros((N_NODES, FPAD - FEAT - 1), jnp.float32)
  xfull = jnp.concatenate([x, ones, zcols], axis=1)
  xfull = jnp.concatenate([xfull, jnp.zeros((8, FPAD), jnp.float32)], axis=0)
  zrows = jnp.zeros((ZROWS, FPAD), jnp.float32)

  agg = _sc_aggregate(xfull, src_p, dst_p, zrows)

  w2p = jnp.zeros((FPAD, FEAT), jnp.float32).at[:FEAT, :].set(W)
  return _tc_combine(x, agg, w2p, loop_weight, b.reshape(1, FEAT))


# speedup vs baseline: 6.3491x; 1.0251x over previous
import functools

import jax
import jax.numpy as jnp
from jax import lax
from jax.experimental import pallas as pl
from jax.experimental.pallas import tpu as pltpu
from jax.experimental.pallas import tpu_sc as plsc

N_NODES = 10000
N_EDGES = 160000
FEAT = 256
FPAD = 272
NSC = 2
NSUB = 16
NHALF = N_NODES // NSC
EPT = 10240
EPAD = NSUB * EPT
CHUNK = 16
NBUF = 5
IDXB = EPT + NBUF * CHUNK + 16
ZROWS = 316
AGG_ROWS = NSUB * ZROWS
DUMMY_SRC = N_NODES
DUMMY_DST = NHALF


def _sc_aggregate(xfull, src_p, dst_p, zrows):
  mesh = plsc.VectorSubcoreMesh(
      core_axis_name="c", subcore_axis_name="s",
      num_cores=NSC, num_subcores=NSUB)

  @functools.partial(
      pl.kernel,
      out_type=jax.ShapeDtypeStruct((NSC, AGG_ROWS, FPAD), jnp.float32),
      mesh=mesh,
      compiler_params=pltpu.CompilerParams(
          use_tc_tiling_on_sc=False, needs_layout_passes=False),
      scratch_types=[
          pltpu.VMEM_SHARED((AGG_ROWS, FPAD), jnp.float32),
          pltpu.VMEM((IDXB,), jnp.int32),
          pltpu.VMEM((IDXB,), jnp.int32),
          [pltpu.VMEM((CHUNK, FPAD), jnp.float32)] * NBUF,
          [pltpu.SemaphoreType.DMA] * NBUF,
      ],
  )
  def agg_kernel(xfull_hbm, src_hbm, dst_hbm, z_hbm, out_hbm,
                 acc, srcb, dstb, rows, sems):
    c = lax.axis_index("c")
    s = lax.axis_index("s")
    lo = c * NHALF

    pltpu.sync_copy(z_hbm, acc.at[pl.ds(s * ZROWS, ZROWS)])

    pltpu.sync_copy(src_hbm.at[s], srcb.at[pl.ds(0, EPT)])
    pltpu.sync_copy(dst_hbm.at[s], dstb.at[pl.ds(0, EPT)])

    @pl.loop(0, EPT // 16, init_carry=jnp.int32(0))
    def _filter(i, off):
      sl = pl.ds(i * 16, 16)
      d = dstb[sl]
      sv = srcb[sl]
      msk = (d >= lo) & (d < lo + NHALF)
      plsc.store_compressed(dstb.at[pl.ds(off, 16)], d - lo, mask=msk)
      plsc.store_compressed(srcb.at[pl.ds(off, 16)], sv, mask=msk)
      cnt = plsc.all_reduce_population_count(msk)
      return off + cnt[0]

    m = _filter
    base = pl.multiple_of((m // 16) * 16, 16)
    lanes = lax.iota(jnp.int32, 16)
    keep = lanes < (m - base)
    bsl = pl.ds(base, 16)
    srcb[bsl] = jnp.where(keep, srcb[bsl], DUMMY_SRC)
    dstb[bsl] = jnp.where(keep, dstb[bsl], DUMMY_DST)
    for k in range(1, NBUF * CHUNK // 16 + 1):
      sl = pl.ds(base + k * 16, 16)
      srcb[sl] = jnp.full((16,), DUMMY_SRC, jnp.int32)
      dstb[sl] = jnp.full((16,), DUMMY_DST, jnp.int32)
    grp = NBUF * CHUNK
    nch = jnp.maximum(NBUF * ((m + grp - 1) // grp), NBUF)

    plsc.subcore_barrier()

    def _gather(t, k):
      sl = srcb.at[pl.ds(t * CHUNK, CHUNK)]
      pltpu.async_copy(xfull_hbm.at[sl], rows[k], sems[k])

    for k in range(NBUF - 1):
      _gather(k, k)

    @pl.loop(0, nch, step=NBUF)
    def _(j):
      for k in range(NBUF):
        t = j + k
        sl = srcb.at[pl.ds(t * CHUNK, CHUNK)]
        pltpu.make_async_copy(xfull_hbm.at[sl], rows[k], sems[k]).wait()

        @pl.when(t + NBUF - 1 < nch)
        def _():
          _gather(t + NBUF - 1, (k + NBUF - 1) % NBUF)

        dv = dstb[pl.ds(t * CHUNK, 16)]
        pltpu.sync_copy(rows[k], acc.at[dv], add=True)

    plsc.subcore_barrier()
    pltpu.sync_copy(acc.at[pl.ds(s * ZROWS, ZROWS)],
                    out_hbm.at[c, pl.ds(s * ZROWS, ZROWS)])

  return agg_kernel(xfull, src_p, dst_p, zrows)


def _tc_body(x_ref, agg_ref, w_ref, lw_ref, b_ref, out_ref):
  a = agg_ref[0]
  deg = jnp.maximum(a[:, FEAT:FEAT + 1], 1.0)
  inv = 1.0 / deg
  acc = jnp.dot(a[:, :FEAT] * inv, w_ref[...],
                preferred_element_type=jnp.float32)
  acc = acc + jnp.dot(x_ref[...], lw_ref[...], preferred_element_type=jnp.float32)
  out_ref[...] = acc + b_ref[...]


def _tc_combine(x, agg, w2p, lw, b2):
  nblk = 10
  blk = N_NODES // nblk
  bph = NHALF // blk
  return pl.pallas_call(
      _tc_body,
      grid=(nblk,),
      in_specs=[
          pl.BlockSpec((blk, FEAT), lambda i: (i, 0)),
          pl.BlockSpec((1, blk, FPAD), lambda i: (i // bph, i % bph, 0)),
          pl.BlockSpec((FEAT, FEAT), lambda i: (0, 0)),
          pl.BlockSpec((FEAT, FEAT), lambda i: (0, 0)),
          pl.BlockSpec((1, FEAT), lambda i: (0, 0)),
      ],
      out_specs=pl.BlockSpec((blk, FEAT), lambda i: (i, 0)),
      out_shape=jax.ShapeDtypeStruct((N_NODES, FEAT), jnp.float32),
  )(x, agg, w2p, lw, b2)


def kernel(x, edge_index, W, b, loop_weight):
  ei = edge_index.astype(jnp.int32)
  src = ei[0]
  dst = ei[1]
  pad = EPAD - N_EDGES
  src_p = jnp.concatenate([src, jnp.full((pad,), DUMMY_SRC, jnp.int32)])
  src_p = src_p.reshape(NSUB, EPT)
  dst_p = jnp.concatenate([dst, jnp.full((pad,), N_NODES, jnp.int32)])
  dst_p = dst_p.reshape(NSUB, EPT)

  ones = jnp.ones((N_NODES, 1), jnp.float32)
  zcols = jnp.zeros((N_NODES, FPAD - FEAT - 1), jnp.float32)
  xfull = jnp.concatenate([x, ones, zcols], axis=1)
  xfull = jnp.concatenate([xfull, jnp.zeros((8, FPAD), jnp.float32)], axis=0)
  zrows = jnp.zeros((ZROWS, FPAD), jnp.float32)

  agg = _sc_aggregate(xfull, src_p, dst_p, zrows)

  return _tc_combine(x, agg, W, loop_weight, b.reshape(1, FEAT))
